# baseline (speedup 1.0000x reference)
import functools

import jax
import jax.numpy as jnp
from jax import lax
from jax.experimental import pallas as pl
from jax.experimental.pallas import tpu as pltpu
from jax.experimental.pallas import tpu_sc as plsc

B = 2
CD = 512
H = 224
W = 224
P = H * W
NW = 32
WPB = NW // B
PP = P // WPB
CC = 8
NCHUNK = CD // CC
NVEC = PP // 16


def _sc_body(x_hbm, tk_hbm, mn_hbm, buf0, buf1, m1r, m2r, m3r, m4r, sr,
             sem0, sem1):
    wid = lax.axis_index("s") * 2 + lax.axis_index("c")
    bb = wid // WPB
    p0 = (wid % WPB) * PP
    row0 = bb * CD

    @pl.loop(0, NVEC)
    def _init(j):
        sl = pl.ds(j * 16, 16)
        ninf = jnp.full((16,), -jnp.inf, jnp.float32)
        m1r[sl] = ninf
        m2r[sl] = ninf
        m3r[sl] = ninf
        m4r[sl] = ninf
        sr[sl] = jnp.zeros((16,), jnp.float32)

    bufs = (buf0, buf1)
    sems = (sem0, sem1)

    def _start(chunk, buf, sem):
        pltpu.make_async_copy(
            x_hbm.at[pl.ds(row0 + chunk * CC, CC), pl.ds(p0, PP)], buf, sem
        ).start()

    def _wait(buf, sem):
        pltpu.make_async_copy(
            x_hbm.at[pl.ds(row0, CC), pl.ds(p0, PP)], buf, sem
        ).wait()

    _start(0, buf0, sem0)
    _start(1, buf1, sem1)

    def _consume(buf):
        @pl.loop(0, NVEC // 2)
        def _pix(j):
            base = j * 32
            for off in (0, 16):
                sl = pl.ds(base + off, 16)
                m1 = m1r[sl]
                m2 = m2r[sl]
                m3 = m3r[sl]
                m4 = m4r[sl]
                s = sr[sl]
                for c in range(CC):
                    v = buf[c, sl]
                    s = s + v
                    nm1 = jnp.maximum(m1, v)
                    nm2 = jnp.maximum(m2, jnp.minimum(v, m1))
                    nm3 = jnp.maximum(m3, jnp.minimum(v, m2))
                    nm4 = jnp.maximum(m4, jnp.minimum(v, m3))
                    m1, m2, m3, m4 = nm1, nm2, nm3, nm4
                m1r[sl] = m1
                m2r[sl] = m2
                m3r[sl] = m3
                m4r[sl] = m4
                sr[sl] = s

    @pl.loop(0, NCHUNK, step=2)
    def _chunks(g):
        for b in range(2):
            gc = g + b
            _wait(bufs[b], sems[b])
            _consume(bufs[b])

            @pl.when(gc + 2 < NCHUNK)
            def _():
                _start(gc + 2, bufs[b], sems[b])

    @pl.loop(0, NVEC)
    def _fin(j):
        sl = pl.ds(j * 16, 16)
        m1r[sl] = (m1r[sl] + m2r[sl]) + (m3r[sl] + m4r[sl])
        sr[sl] = sr[sl] * jnp.float32(1.0 / CD)

    base = bb * P + p0
    pltpu.sync_copy(m1r, tk_hbm.at[pl.ds(base, PP)])
    pltpu.sync_copy(sr, mn_hbm.at[pl.ds(base, PP)])


def _sc_stage(x2):
    mesh = plsc.VectorSubcoreMesh(core_axis_name="c", subcore_axis_name="s")
    return pl.kernel(
        _sc_body,
        out_type=[
            jax.ShapeDtypeStruct((B * P,), jnp.float32),
            jax.ShapeDtypeStruct((B * P,), jnp.float32),
        ],
        mesh=mesh,
        scratch_types=[
            pltpu.VMEM((CC, PP), jnp.float32),
            pltpu.VMEM((CC, PP), jnp.float32),
            pltpu.VMEM((PP,), jnp.float32),
            pltpu.VMEM((PP,), jnp.float32),
            pltpu.VMEM((PP,), jnp.float32),
            pltpu.VMEM((PP,), jnp.float32),
            pltpu.VMEM((PP,), jnp.float32),
            pltpu.SemaphoreType.DMA,
            pltpu.SemaphoreType.DMA,
        ],
        compiler_params=pltpu.CompilerParams(use_tc_tiling_on_sc=False),
    )(x2)


def _conv_body(a1_ref, a2_ref, w1_ref, b1_ref, w2_ref, b2_ref, att_ref):
    a1 = a1_ref[0]
    a2 = a2_ref[0]
    acc = jnp.full((H, W), b1_ref[0], jnp.float32)
    for ci, a in enumerate((a1, a2)):
        p = jnp.pad(a, 1)
        for di in range(3):
            for dj in range(3):
                acc = acc + w1_ref[ci, di, dj] * p[di:di + H, dj:dj + W]
    m = jnp.maximum(acc, 0.0)
    acc2 = jnp.full((H, W), b2_ref[0], jnp.float32)
    pm = jnp.pad(m, 1)
    for di in range(3):
        for dj in range(3):
            acc2 = acc2 + w2_ref[di, dj] * pm[di:di + H, dj:dj + W]
    att_ref[0] = jax.nn.sigmoid(acc2)


def _conv_stage(a1, a2, w1m, b1, w2m, b2):
    return pl.pallas_call(
        _conv_body,
        grid=(B,),
        in_specs=[
            pl.BlockSpec((1, H, W), lambda b: (b, 0, 0)),
            pl.BlockSpec((1, H, W), lambda b: (b, 0, 0)),
            pl.BlockSpec(memory_space=pltpu.SMEM),
            pl.BlockSpec(memory_space=pltpu.SMEM),
            pl.BlockSpec(memory_space=pltpu.SMEM),
            pl.BlockSpec(memory_space=pltpu.SMEM),
        ],
        out_specs=pl.BlockSpec((1, H, W), lambda b: (b, 0, 0)),
        out_shape=jax.ShapeDtypeStruct((B, H, W), jnp.float32),
    )(a1, a2, w1m, b1, w2m, b2)


CK = 8


def _mul_body(x_ref, att_ref, o_ref):
    o_ref[...] = x_ref[...] * att_ref[...]


def _mul_stage(x3, att3):
    return pl.pallas_call(
        _mul_body,
        grid=(B, CD // CK),
        in_specs=[
            pl.BlockSpec((1, CK, P), lambda b, c: (b, c, 0)),
            pl.BlockSpec((1, 1, P), lambda b, c: (b, 0, 0)),
        ],
        out_specs=pl.BlockSpec((1, CK, P), lambda b, c: (b, c, 0)),
        out_shape=jax.ShapeDtypeStruct((B, CD, P), jnp.float32),
    )(x3, att3)


@jax.jit
def kernel(x, W1, b1, W2, b2):
    x2 = x.reshape(B * CD, P)
    tk, mn = _sc_stage(x2)
    a1 = tk.reshape(B, H, W)
    a2 = mn.reshape(B, H, W)
    w1m = W1[0, :, 1]
    w2m = W2[0, 0, 1]
    att = _conv_stage(a1, a2, w1m, b1, w2m, b2)
    out3 = _mul_stage(x.reshape(B, CD, P), att.reshape(B, 1, P))
    return (
        out3.reshape(x.shape),
        att.reshape(B, 1, 1, H, W),
    )


# speedup vs baseline: 1.4775x; 1.4775x over previous
import functools

import jax
import jax.numpy as jnp
from jax import lax
from jax.experimental import pallas as pl
from jax.experimental.pallas import tpu as pltpu
from jax.experimental.pallas import tpu_sc as plsc

B = 2
CD = 512
H = 224
W = 224
P = H * W
NW = 32
WPB = NW // B
PP = P // WPB
CC = 8
NCHUNK = CD // CC
NVEC = PP // 16


def _sc_body(x_hbm, tk_hbm, mn_hbm, buf0, buf1, m1r, m2r, m3r, m4r, sr,
             sem0, sem1):
    wid = lax.axis_index("s") * 2 + lax.axis_index("c")
    bb = wid // WPB
    p0 = (wid % WPB) * PP
    row0 = bb * CD

    @pl.loop(0, NVEC)
    def _init(j):
        sl = pl.ds(j * 16, 16)
        ninf = jnp.full((16,), -jnp.inf, jnp.float32)
        m1r[sl] = ninf
        m2r[sl] = ninf
        m3r[sl] = ninf
        m4r[sl] = ninf
        sr[sl] = jnp.zeros((16,), jnp.float32)

    bufs = (buf0, buf1)
    sems = (sem0, sem1)

    def _start(chunk, buf, sem):
        pltpu.make_async_copy(
            x_hbm.at[pl.ds(row0 + chunk * CC, CC), pl.ds(p0, PP)], buf, sem
        ).start()

    def _wait(buf, sem):
        pltpu.make_async_copy(
            x_hbm.at[pl.ds(row0, CC), pl.ds(p0, PP)], buf, sem
        ).wait()

    _start(0, buf0, sem0)
    _start(1, buf1, sem1)

    def _consume(buf):
        @pl.loop(0, NVEC // 2)
        def _pix(j):
            base = j * 32
            for off in (0, 16):
                sl = pl.ds(base + off, 16)
                m1 = m1r[sl]
                m2 = m2r[sl]
                m3 = m3r[sl]
                m4 = m4r[sl]
                s = sr[sl]
                for c in range(CC):
                    v = buf[c, sl]
                    s = s + v
                    nm1 = jnp.maximum(m1, v)
                    nm2 = jnp.maximum(m2, jnp.minimum(v, m1))
                    nm3 = jnp.maximum(m3, jnp.minimum(v, m2))
                    nm4 = jnp.maximum(m4, jnp.minimum(v, m3))
                    m1, m2, m3, m4 = nm1, nm2, nm3, nm4
                m1r[sl] = m1
                m2r[sl] = m2
                m3r[sl] = m3
                m4r[sl] = m4
                sr[sl] = s

    @pl.loop(0, NCHUNK, step=2)
    def _chunks(g):
        for b in range(2):
            gc = g + b
            _wait(bufs[b], sems[b])
            _consume(bufs[b])

            @pl.when(gc + 2 < NCHUNK)
            def _():
                _start(gc + 2, bufs[b], sems[b])

    @pl.loop(0, NVEC)
    def _fin(j):
        sl = pl.ds(j * 16, 16)
        m1r[sl] = (m1r[sl] + m2r[sl]) + (m3r[sl] + m4r[sl])
        sr[sl] = sr[sl] * jnp.float32(1.0 / CD)

    base = bb * P + p0
    pltpu.sync_copy(m1r, tk_hbm.at[pl.ds(base, PP)])
    pltpu.sync_copy(sr, mn_hbm.at[pl.ds(base, PP)])


def _sc_stage(x2):
    mesh = plsc.VectorSubcoreMesh(core_axis_name="c", subcore_axis_name="s")
    return pl.kernel(
        _sc_body,
        out_type=[
            jax.ShapeDtypeStruct((B * P,), jnp.float32),
            jax.ShapeDtypeStruct((B * P,), jnp.float32),
        ],
        mesh=mesh,
        scratch_types=[
            pltpu.VMEM((CC, PP), jnp.float32),
            pltpu.VMEM((CC, PP), jnp.float32),
            pltpu.VMEM((PP,), jnp.float32),
            pltpu.VMEM((PP,), jnp.float32),
            pltpu.VMEM((PP,), jnp.float32),
            pltpu.VMEM((PP,), jnp.float32),
            pltpu.VMEM((PP,), jnp.float32),
            pltpu.SemaphoreType.DMA,
            pltpu.SemaphoreType.DMA,
        ],
        compiler_params=pltpu.CompilerParams(use_tc_tiling_on_sc=False),
    )(x2)


def _conv_body(a1_ref, a2_ref, w1_ref, b1_ref, w2_ref, b2_ref, att_ref):
    a1 = a1_ref[0]
    a2 = a2_ref[0]
    acc = jnp.full((H, W), b1_ref[0], jnp.float32)
    for ci, a in enumerate((a1, a2)):
        p = jnp.pad(a, 1)
        for di in range(3):
            for dj in range(3):
                acc = acc + w1_ref[ci, di, dj] * p[di:di + H, dj:dj + W]
    m = jnp.maximum(acc, 0.0)
    acc2 = jnp.full((H, W), b2_ref[0], jnp.float32)
    pm = jnp.pad(m, 1)
    for di in range(3):
        for dj in range(3):
            acc2 = acc2 + w2_ref[di, dj] * pm[di:di + H, dj:dj + W]
    att_ref[0] = jax.nn.sigmoid(acc2)


def _conv_stage(a1, a2, w1m, b1, w2m, b2):
    return pl.pallas_call(
        _conv_body,
        grid=(B,),
        in_specs=[
            pl.BlockSpec((1, H, W), lambda b: (b, 0, 0)),
            pl.BlockSpec((1, H, W), lambda b: (b, 0, 0)),
            pl.BlockSpec(memory_space=pltpu.SMEM),
            pl.BlockSpec(memory_space=pltpu.SMEM),
            pl.BlockSpec(memory_space=pltpu.SMEM),
            pl.BlockSpec(memory_space=pltpu.SMEM),
        ],
        out_specs=pl.BlockSpec((1, H, W), lambda b: (b, 0, 0)),
        out_shape=jax.ShapeDtypeStruct((B, H, W), jnp.float32),
    )(a1, a2, w1m, b1, w2m, b2)


def _mul_body(x_ref, att_ref, o_ref):
    o_ref[...] = x_ref[...] * att_ref[0][None, None]


def _mul_stage(x, att):
    return pl.pallas_call(
        _mul_body,
        grid=(B, 32),
        in_specs=[
            pl.BlockSpec((1, 1, 16, H, W), lambda b, c: (b, c, 0, 0, 0)),
            pl.BlockSpec((1, H, W), lambda b, c: (b, 0, 0)),
        ],
        out_specs=pl.BlockSpec((1, 1, 16, H, W), lambda b, c: (b, c, 0, 0, 0)),
        out_shape=jax.ShapeDtypeStruct((B, 32, 16, H, W), jnp.float32),
    )(x, att)


@jax.jit
def kernel(x, W1, b1, W2, b2):
    x2 = x.reshape(B * CD, P)
    tk, mn = _sc_stage(x2)
    a1 = tk.reshape(B, H, W)
    a2 = mn.reshape(B, H, W)
    w1m = W1[0, :, 1]
    w2m = W2[0, 0, 1]
    att = _conv_stage(a1, a2, w1m, b1, w2m, b2)
    out5 = _mul_stage(x, att)
    return (
        out5.reshape(x.shape),
        att.reshape(B, 1, 1, H, W),
    )
